# baseline (speedup 1.0000x reference)
import functools
import jax
import jax.numpy as jnp
from jax import lax
from jax.experimental import pallas as pl
from jax.experimental.pallas import tpu as pltpu
from jax.experimental.pallas import tpu_sc as plsc

VOCAB = 1000000
EMBED_DIM = 64
BATCH = 16384
HIST = 50

NC = 2
NS = 16
NW = NC * NS

CB = 128
KPW = BATCH // NW // CB
UNITS = HIST * KPW

_mesh = plsc.VectorSubcoreMesh(
    core_axis_name="c", subcore_axis_name="s", num_cores=NC, num_subcores=NS
)


@functools.partial(
    pl.kernel,
    out_type=jax.ShapeDtypeStruct((HIST, 8, BATCH // CB, 8, CB), jnp.float32),
    mesh=_mesh,
    scratch_types=[
        pltpu.VMEM((HIST, KPW, CB), jnp.int32),
        pltpu.VMEM((2, CB, EMBED_DIM), jnp.float32),
        pltpu.VMEM((2, 8, 8, CB), jnp.float32),
        [pltpu.SemaphoreType.DMA] * 2,
        [pltpu.SemaphoreType.DMA] * 2,
    ],
    compiler_params=pltpu.CompilerParams(
        use_tc_tiling_on_sc=False, needs_layout_passes=False
    ),
)
def _gather_kernel(xt_hbm, table_hbm, out_hbm, idx_v, g_v, t_v, gsems, wsems):
    wid = lax.axis_index("s") * NC + lax.axis_index("c")
    pltpu.sync_copy(xt_hbm.at[:, pl.ds(wid * KPW, KPW), :], idx_v)

    def unit_hk(u):
        return u // KPW, lax.rem(u, KPW)

    def fire_gather(u, slot):
        h, k = unit_hk(u)
        pltpu.async_copy(
            table_hbm.at[idx_v.at[h, k]], g_v.at[slot], gsems[slot]
        )

    def out_ref_for(u):
        h, k = unit_hk(u)
        return out_hbm.at[h, :, wid * KPW + k]

    lane = lax.broadcasted_iota(jnp.int32, (16,), 0)

    def do_unit(u, slot):
        pltpu.make_async_copy(
            table_hbm.at[idx_v.at[unit_hk(u)[0], unit_hk(u)[1]]],
            g_v.at[slot],
            gsems[slot],
        ).wait()

        @pl.when(u >= 2)
        def _():
            pltpu.make_async_copy(
                t_v.at[slot], out_ref_for(u - 2), wsems[slot]
            ).wait()

        def per_td(td, carry):
            dbase = jnp.full((16,), td * 8, jnp.int32)
            for dr in range(8):
                dcol = dbase + dr
                for c in range(CB // 16):
                    v = plsc.load_gather(
                        g_v.at[slot], [lane + (16 * c), dcol]
                    )
                    t_v[slot, td, dr, pl.ds(16 * c, 16)] = v
            return carry

        lax.fori_loop(0, 8, per_td, 0)

        pltpu.async_copy(t_v.at[slot], out_ref_for(u), wsems[slot])

        @pl.when(u + 2 < UNITS)
        def _():
            fire_gather(u + 2, slot)

    fire_gather(0, 0)
    fire_gather(1, 1)

    def pair(p, carry):
        do_unit(2 * p, 0)
        do_unit(2 * p + 1, 1)
        return carry

    lax.fori_loop(0, UNITS // 2, pair, 0)

    for slot, u in ((0, UNITS - 2), (1, UNITS - 1)):
        pltpu.make_async_copy(t_v.at[slot], out_ref_for(u), wsems[slot]).wait()


def kernel(X, table):
    xt = X.T.reshape(HIST, BATCH // CB, CB).astype(jnp.int32)
    p = _gather_kernel(xt, table)
    out = p.transpose(2, 4, 0, 1, 3).reshape(BATCH, HIST, EMBED_DIM)
    return out


# speedup vs baseline: 1.2663x; 1.2663x over previous
import functools
import jax
import jax.numpy as jnp
from jax import lax
from jax.experimental import pallas as pl
from jax.experimental.pallas import tpu as pltpu
from jax.experimental.pallas import tpu_sc as plsc

VOCAB = 1000000
EMBED_DIM = 64
BATCH = 16384
HIST = 50

NC = 2
NS = 16
NW = NC * NS

CB = 128
KPW = BATCH // NW // CB
UNITS = HIST * KPW

_mesh = plsc.VectorSubcoreMesh(
    core_axis_name="c", subcore_axis_name="s", num_cores=NC, num_subcores=NS
)


@functools.partial(
    pl.kernel,
    out_type=jax.ShapeDtypeStruct((HIST, 8, BATCH // CB, 8, CB), jnp.float32),
    mesh=_mesh,
    scratch_types=[
        pltpu.VMEM((HIST, KPW, CB), jnp.int32),
        pltpu.VMEM((2, CB, EMBED_DIM), jnp.float32),
        pltpu.VMEM((2, 8, 8, CB), jnp.float32),
        [pltpu.SemaphoreType.DMA] * 2,
        [pltpu.SemaphoreType.DMA] * 2,
    ],
    compiler_params=pltpu.CompilerParams(
        use_tc_tiling_on_sc=False, needs_layout_passes=False
    ),
)
def _gather_kernel(xt_hbm, table_hbm, out_hbm, idx_v, g_v, t_v, gsems, wsems):
    wid = lax.axis_index("s") * NC + lax.axis_index("c")
    pltpu.sync_copy(xt_hbm.at[:, pl.ds(wid * KPW, KPW), :], idx_v)

    def unit_hk(u):
        return u // KPW, lax.rem(u, KPW)

    def fire_gather(u, slot):
        h, k = unit_hk(u)
        pltpu.async_copy(
            table_hbm.at[idx_v.at[h, k]], g_v.at[slot], gsems[slot]
        )

    def out_ref_for(u):
        h, k = unit_hk(u)
        return out_hbm.at[h, :, wid * KPW + k]

    lane = lax.broadcasted_iota(jnp.int32, (16,), 0)

    def do_unit(u, slot):
        pltpu.make_async_copy(
            table_hbm.at[idx_v.at[unit_hk(u)[0], unit_hk(u)[1]]],
            g_v.at[slot],
            gsems[slot],
        ).wait()

        @pl.when(u >= 2)
        def _():
            pltpu.make_async_copy(
                t_v.at[slot], out_ref_for(u - 2), wsems[slot]
            ).wait()

        @plsc.parallel_loop(0, 8, unroll=2)
        def _(td):
            dbase = jnp.full((16,), td * 8, jnp.int32)
            for dr in range(8):
                dcol = dbase + dr
                for c in range(CB // 16):
                    v = plsc.load_gather(
                        g_v.at[slot], [lane + (16 * c), dcol]
                    )
                    t_v[slot, td, dr, pl.ds(16 * c, 16)] = v

        pltpu.async_copy(t_v.at[slot], out_ref_for(u), wsems[slot])

        @pl.when(u + 2 < UNITS)
        def _():
            fire_gather(u + 2, slot)

    fire_gather(0, 0)
    fire_gather(1, 1)

    def pair(p, carry):
        do_unit(2 * p, 0)
        do_unit(2 * p + 1, 1)
        return carry

    lax.fori_loop(0, UNITS // 2, pair, 0)

    for slot, u in ((0, UNITS - 2), (1, UNITS - 1)):
        pltpu.make_async_copy(t_v.at[slot], out_ref_for(u), wsems[slot]).wait()


def kernel(X, table):
    xt = X.T.reshape(HIST, BATCH // CB, CB).astype(jnp.int32)
    p = _gather_kernel(xt, table)
    out = p.transpose(2, 4, 0, 1, 3).reshape(BATCH, HIST, EMBED_DIM)
    return out


# speedup vs baseline: 1.4330x; 1.1317x over previous
import functools
import jax
import jax.numpy as jnp
from jax import lax
from jax.experimental import pallas as pl
from jax.experimental.pallas import tpu as pltpu
from jax.experimental.pallas import tpu_sc as plsc

VOCAB = 1000000
EMBED_DIM = 64
BATCH = 16384
HIST = 50

NC = 2
NS = 16
NW = NC * NS

CB = 128
KPW = BATCH // NW // CB
UNITS = HIST * KPW

_mesh = plsc.VectorSubcoreMesh(
    core_axis_name="c", subcore_axis_name="s", num_cores=NC, num_subcores=NS
)


@functools.partial(
    pl.kernel,
    out_type=jax.ShapeDtypeStruct((HIST, 8, BATCH // CB, 8, CB), jnp.float32),
    mesh=_mesh,
    scratch_types=[
        pltpu.VMEM((HIST, KPW, CB), jnp.int32),
        pltpu.VMEM((2, CB, EMBED_DIM), jnp.float32),
        pltpu.VMEM((2, 8, 8, CB), jnp.float32),
        [pltpu.SemaphoreType.DMA] * 2,
        [pltpu.SemaphoreType.DMA] * 2,
    ],
    compiler_params=pltpu.CompilerParams(
        use_tc_tiling_on_sc=False, needs_layout_passes=False
    ),
)
def _gather_kernel(xt_hbm, table_hbm, out_hbm, idx_v, g_v, t_v, gsems, wsems):
    wid = lax.axis_index("s") * NC + lax.axis_index("c")
    pltpu.sync_copy(xt_hbm.at[:, pl.ds(wid * KPW, KPW), :], idx_v)

    def unit_hk(u):
        return u // KPW, lax.rem(u, KPW)

    def fire_gather(u, slot):
        h, k = unit_hk(u)
        pltpu.async_copy(
            table_hbm.at[idx_v.at[h, k]], g_v.at[slot], gsems[slot]
        )

    def out_ref_for(u):
        h, k = unit_hk(u)
        return out_hbm.at[h, :, wid * KPW + k]

    lane = lax.broadcasted_iota(jnp.int32, (16,), 0)

    def do_unit(u, slot):
        pltpu.make_async_copy(
            table_hbm.at[idx_v.at[unit_hk(u)[0], unit_hk(u)[1]]],
            g_v.at[slot],
            gsems[slot],
        ).wait()

        @pl.when(u >= 2)
        def _():
            pltpu.make_async_copy(
                t_v.at[slot], out_ref_for(u - 2), wsems[slot]
            ).wait()

        @plsc.parallel_loop(0, 8, unroll=4)
        def _(td):
            dbase = jnp.full((16,), td * 8, jnp.int32)
            for dr in range(8):
                dcol = dbase + dr
                for c in range(CB // 16):
                    v = plsc.load_gather(
                        g_v.at[slot], [lane + (16 * c), dcol]
                    )
                    t_v[slot, td, dr, pl.ds(16 * c, 16)] = v

        pltpu.async_copy(t_v.at[slot], out_ref_for(u), wsems[slot])

        @pl.when(u + 2 < UNITS)
        def _():
            fire_gather(u + 2, slot)

    fire_gather(0, 0)
    fire_gather(1, 1)

    def pair(p, carry):
        do_unit(2 * p, 0)
        do_unit(2 * p + 1, 1)
        return carry

    lax.fori_loop(0, UNITS // 2, pair, 0)

    for slot, u in ((0, UNITS - 2), (1, UNITS - 1)):
        pltpu.make_async_copy(t_v.at[slot], out_ref_for(u), wsems[slot]).wait()


def kernel(X, table):
    xt = X.T.reshape(HIST, BATCH // CB, CB).astype(jnp.int32)
    p = _gather_kernel(xt, table)
    out = p.transpose(2, 4, 0, 1, 3).reshape(BATCH, HIST, EMBED_DIM)
    return out
